# Initial kernel scaffold; baseline (speedup 1.0000x reference)
import jax
import jax.numpy as jnp
from jax.experimental import pallas as pl


def kernel(x, tok_table):
    raise NotImplementedError("write your pallas kernel here")


# speedup vs baseline: 1.2891x; 1.2891x over previous
import functools

import numpy as np
import jax
import jax.numpy as jnp
from jax import lax
from jax.experimental import pallas as pl
from jax.experimental.pallas import tpu as pltpu
from jax.experimental.pallas import tpu_sc as plsc


@functools.lru_cache(maxsize=None)
def _pos_encoding(seq_len: int, d_model: int):
    pos = np.arange(0, seq_len, dtype=np.float32)[:, None]
    p_2i = np.arange(0, d_model, 2, dtype=np.float32)
    div = np.power(np.float32(10000.0), p_2i / np.float32(d_model))
    enc = np.zeros((seq_len, d_model), dtype=np.float32)
    enc[:, 0::2] = np.sin(pos / div)
    enc[:, 1::2] = np.cos(pos / div)
    return jnp.asarray(enc)


@functools.lru_cache(maxsize=None)
def _build_sc_kernel(B: int, S: int, V: int, D: int):
    info = plsc.get_sparse_core_info()
    NC, NS, L = info.num_cores, info.num_subcores, info.num_lanes
    NW = NC * NS
    SW = S // NW
    C = 16
    NCH = SW // C
    NVEC = D // L

    mesh = plsc.VectorSubcoreMesh(core_axis_name="c", subcore_axis_name="s")

    @functools.partial(
        pl.kernel,
        out_type=jax.ShapeDtypeStruct((B * S, D), jnp.float32),
        mesh=mesh,
        scratch_types=[
            pltpu.VMEM((B, SW), jnp.int32),
            pltpu.VMEM((C, D), jnp.float32),
            pltpu.VMEM((C, D), jnp.float32),
            pltpu.SemaphoreType.DMA,
        ],
    )
    def k(x_hbm, enc_hbm, table_hbm, out_hbm, idx_v, enc_v, rows_v, sem):
        wid = lax.axis_index("s") * NC + lax.axis_index("c")
        s_base = wid * SW
        for b in range(B):
            pltpu.sync_copy(x_hbm.at[b, pl.ds(s_base, SW)], idx_v.at[b])

        def chunk_body(g, carry):
            s0 = s_base + g * C
            pltpu.sync_copy(enc_hbm.at[pl.ds(s0, C)], enc_v)
            for b in range(B):
                pltpu.async_copy(
                    table_hbm.at[idx_v.at[b, pl.ds(g * C, C)]], rows_v, sem
                ).wait()

                def row_body(i, c2):
                    for j in range(NVEC):
                        sl = pl.ds(j * L, L)
                        rows_v[i, sl] = rows_v[i, sl] + enc_v[i, sl]
                    return c2

                lax.fori_loop(0, C, row_body, 0)
                pltpu.sync_copy(rows_v, out_hbm.at[pl.ds(b * S + s0, C)])
            return carry

        lax.fori_loop(0, NCH, chunk_body, 0)

    return k


def kernel(x, tok_table):
    B, S = x.shape
    V, D = tok_table.shape
    enc = _pos_encoding(S, D)
    out = _build_sc_kernel(B, S, V, D)(x.astype(jnp.int32), enc, tok_table)
    return out.reshape(B, S, D)


# speedup vs baseline: 2.4737x; 1.9189x over previous
import functools

import numpy as np
import jax
import jax.numpy as jnp
from jax import lax
from jax.experimental import pallas as pl
from jax.experimental.pallas import tpu as pltpu
from jax.experimental.pallas import tpu_sc as plsc


@functools.lru_cache(maxsize=None)
def _pos_encoding(seq_len: int, d_model: int):
    pos = np.arange(0, seq_len, dtype=np.float32)[:, None]
    p_2i = np.arange(0, d_model, 2, dtype=np.float32)
    div = np.power(np.float32(10000.0), p_2i / np.float32(d_model))
    enc = np.zeros((seq_len, d_model), dtype=np.float32)
    enc[:, 0::2] = np.sin(pos / div)
    enc[:, 1::2] = np.cos(pos / div)
    return jnp.asarray(enc)


@functools.lru_cache(maxsize=None)
def _build_sc_kernel(B: int, S: int, V: int, D: int):
    info = plsc.get_sparse_core_info()
    NC, NS, L = info.num_cores, info.num_subcores, info.num_lanes
    NW = NC * NS
    SW = S // NW
    C = 8
    NCH = SW // C
    NB = 2
    NVEC = D // L

    mesh = plsc.VectorSubcoreMesh(core_axis_name="c", subcore_axis_name="s")

    @functools.partial(
        pl.kernel,
        out_type=jax.ShapeDtypeStruct((B * S, D), jnp.float32),
        mesh=mesh,
        scratch_types=[
            pltpu.VMEM((B, SW), jnp.int32),
            pltpu.VMEM((NB, C, D), jnp.float32),
            pltpu.VMEM((NB, B, C, D), jnp.float32),
            pltpu.SemaphoreType.DMA((NB,)),
            pltpu.SemaphoreType.DMA((NB,)),
        ],
    )
    def k(x_hbm, enc_hbm, table_hbm, out_hbm, idx_v, enc_v, rows_v, gsem, ssem):
        wid = lax.axis_index("s") * NC + lax.axis_index("c")
        s_base = wid * SW
        for b in range(B):
            pltpu.sync_copy(x_hbm.at[b, pl.ds(s_base, SW)], idx_v.at[b])

        def in_copies(g, p):
            s0 = s_base + g * C
            yield pltpu.make_async_copy(
                enc_hbm.at[pl.ds(s0, C)], enc_v.at[p], gsem.at[p])
            for b in range(B):
                yield pltpu.make_async_copy(
                    table_hbm.at[idx_v.at[b, pl.ds(g * C, C)]],
                    rows_v.at[p, b], gsem.at[p])

        def out_copies(g, p):
            s0 = s_base + g * C
            for b in range(B):
                yield pltpu.make_async_copy(
                    rows_v.at[p, b], out_hbm.at[pl.ds(b * S + s0, C)],
                    ssem.at[p])

        for c in in_copies(0, 0):
            c.start()

        def step(g, p):
            @pl.when(g >= 1)
            def _():
                for c in out_copies(g - 1, (p + 1) % NB):
                    c.wait()

            @pl.when(g + 1 < NCH)
            def _():
                for c in in_copies(g + 1, (p + 1) % NB):
                    c.start()

            for c in in_copies(g, p):
                c.wait()

            def row_body(i, carry):
                for j in range(NVEC):
                    sl = pl.ds(j * L, L)
                    e = enc_v[p, i, sl]
                    for b in range(B):
                        rows_v[p, b, i, sl] = rows_v[p, b, i, sl] + e
                return carry

            lax.fori_loop(0, C, row_body, 0)

            for c in out_copies(g, p):
                c.start()

        def outer(t, carry):
            for p in range(NB):
                step(t * NB + p, p)
            return carry

        lax.fori_loop(0, NCH // NB, outer, 0)

        for c in out_copies(NCH - 1, (NCH - 1) % NB):
            c.wait()

    return k


def kernel(x, tok_table):
    B, S = x.shape
    V, D = tok_table.shape
    enc = _pos_encoding(S, D)
    out = _build_sc_kernel(B, S, V, D)(x.astype(jnp.int32), enc, tok_table)
    return out.reshape(B, S, D)
